# Initial kernel scaffold; baseline (speedup 1.0000x reference)
import jax
import jax.numpy as jnp
from jax.experimental import pallas as pl


def kernel(entity_emb, rel_emb, W_in, W_loop, skip_weights, edge_index, edge_type):
    raise NotImplementedError("write your pallas kernel here")


# speedup vs baseline: 6.5177x; 6.5177x over previous
import functools

import jax
import jax.numpy as jnp
from jax import lax
from jax.experimental import pallas as pl
from jax.experimental.pallas import tpu as pltpu
from jax.experimental.pallas import tpu_sc as plsc

N_NODES = 10000
N_EDGES = 320000
N_REL = 64
HIDDEN = 128
N_LAYERS = 3

NUM_CORES = 2
NUM_SUBCORES = 16
NW = NUM_CORES * NUM_SUBCORES

CHUNK = 128
TOTAL_CHUNKS = N_EDGES // CHUNK
BASE_CHUNKS = TOTAL_CHUNKS // NW
EXTRA_W = TOTAL_CHUNKS % NW

N_PAD = 10240
ROWS_PER_SUB = N_PAD // NUM_SUBCORES
ZROWS = 128

CNT_TOT = N_NODES * N_REL
CNT_SUBS = 8
CNT_PER_SUB = CNT_TOT // CNT_SUBS
ZCNT = 16000

_mesh = plsc.VectorSubcoreMesh(core_axis_name="c", subcore_axis_name="s")


@functools.partial(
    pl.kernel,
    out_type=jax.ShapeDtypeStruct((NUM_CORES * CNT_TOT,), jnp.float32),
    mesh=_mesh,
    scratch_types=[
        pltpu.VMEM_SHARED((CNT_TOT,), jnp.float32),
        pltpu.VMEM((CHUNK,), jnp.int32),
        pltpu.VMEM((CHUNK,), jnp.int32),
        pltpu.VMEM((CHUNK,), jnp.int32),
        pltpu.VMEM((CHUNK,), jnp.float32),
        pltpu.VMEM((ZCNT,), jnp.float32),
    ],
)
def _sc_hist(dst_hbm, et_hbm, cnt_out, cnt_sh, dst_v, et_v, flat_v, ones_v, zero_v):
    cid = lax.axis_index("c")
    sid = lax.axis_index("s")
    wid = sid * NUM_CORES + cid

    def zstore(i, carry):
        zero_v[pl.ds(i * 16, 16)] = jnp.zeros((16,), jnp.float32)
        return carry
    lax.fori_loop(0, ZCNT // 16, zstore, 0)
    for k in range(CHUNK // 16):
        ones_v[pl.ds(k * 16, 16)] = jnp.ones((16,), jnp.float32)

    @pl.when(sid < CNT_SUBS)
    def _():
        for k in range(CNT_PER_SUB // ZCNT):
            pltpu.sync_copy(zero_v, cnt_sh.at[pl.ds(sid * CNT_PER_SUB + k * ZCNT, ZCNT)])
    plsc.subcore_barrier()

    nj = BASE_CHUNKS + jnp.where(wid < EXTRA_W, 1, 0)

    def chunk_body(t, carry):
        off = pl.multiple_of((wid + t * NW) * CHUNK, CHUNK)
        pltpu.sync_copy(dst_hbm.at[pl.ds(off, CHUNK)], dst_v)
        pltpu.sync_copy(et_hbm.at[pl.ds(off, CHUNK)], et_v)
        for k in range(CHUNK // 16):
            d16 = dst_v[pl.ds(k * 16, 16)]
            e16 = et_v[pl.ds(k * 16, 16)]
            flat_v[pl.ds(k * 16, 16)] = d16 * N_REL + e16
        pltpu.sync_copy(ones_v, cnt_sh.at[flat_v], add=True)
        return carry
    lax.fori_loop(0, nj, chunk_body, 0)
    plsc.subcore_barrier()

    @pl.when(sid < CNT_SUBS)
    def _():
        for k in range(CNT_PER_SUB // ZCNT):
            base = sid * CNT_PER_SUB + k * ZCNT
            pltpu.sync_copy(cnt_sh.at[pl.ds(base, ZCNT)],
                            cnt_out.at[pl.ds(cid * CNT_TOT + base, ZCNT)])


@functools.partial(
    pl.kernel,
    out_type=jax.ShapeDtypeStruct((NUM_CORES, N_PAD, HIDDEN), jnp.float32),
    mesh=_mesh,
    scratch_types=[
        pltpu.VMEM_SHARED((N_PAD, HIDDEN), jnp.float32),
        pltpu.VMEM((CHUNK,), jnp.int32),
        pltpu.VMEM((CHUNK,), jnp.int32),
        pltpu.VMEM((CHUNK, HIDDEN), jnp.float32),
        pltpu.VMEM((ZROWS, HIDDEN), jnp.float32),
        pltpu.SemaphoreType.DMA,
    ],
)
def _sc_agg(src_hbm, dst_hbm, x_hbm, out_hbm, agg_sh, src_v, dst_v, rows_v, zero_v, sem):
    cid = lax.axis_index("c")
    sid = lax.axis_index("s")
    wid = sid * NUM_CORES + cid

    def zrow(r, carry):
        for c in range(HIDDEN // 16):
            zero_v[r, pl.ds(c * 16, 16)] = jnp.zeros((16,), jnp.float32)
        return carry
    lax.fori_loop(0, ZROWS, zrow, 0)
    for k in range(ROWS_PER_SUB // ZROWS):
        pltpu.sync_copy(zero_v, agg_sh.at[pl.ds(sid * ROWS_PER_SUB + k * ZROWS, ZROWS), :])
    plsc.subcore_barrier()

    nj = BASE_CHUNKS + jnp.where(wid < EXTRA_W, 1, 0)

    def chunk_body(t, carry):
        off = pl.multiple_of((wid + t * NW) * CHUNK, CHUNK)
        pltpu.sync_copy(src_hbm.at[pl.ds(off, CHUNK)], src_v)
        pltpu.sync_copy(dst_hbm.at[pl.ds(off, CHUNK)], dst_v)
        pltpu.async_copy(x_hbm.at[src_v], rows_v, sem).wait()
        pltpu.sync_copy(rows_v, agg_sh.at[dst_v], add=True)
        return carry
    lax.fori_loop(0, nj, chunk_body, 0)
    plsc.subcore_barrier()

    for k in range(ROWS_PER_SUB // ZROWS):
        sl = pl.ds(sid * ROWS_PER_SUB + k * ZROWS, ZROWS)
        pltpu.sync_copy(agg_sh.at[sl, :], out_hbm.at[cid, sl, :])


ROW_BLK = 1000


def _tc_layer_body(sw_ref, pa_ref, pb_ref, ca_ref, cb_ref, rel_ref, x_ref,
                   x0_ref, win_ref, wl_ref, o_ref):
    cnt = ca_ref[...] + cb_ref[...]
    deg = jnp.sum(cnt, axis=1, keepdims=True)
    inv = 1.0 / jnp.maximum(deg, 1.0)
    rel_term = jnp.dot(cnt, rel_ref[...], preferred_element_type=jnp.float32)
    agg = (pa_ref[...] + pb_ref[...] - rel_term) * inv
    h = (jnp.dot(agg, win_ref[...], preferred_element_type=jnp.float32)
         + jnp.dot(x_ref[...], wl_ref[...], preferred_element_type=jnp.float32))
    h = jnp.where(h >= 0, h, 0.2 * h)
    alpha = 1.0 / (1.0 + jnp.exp(-sw_ref[0]))
    o_ref[...] = (1.0 - alpha) * h + alpha * x0_ref[...]


def _tc_layer(pa, pb, ca, cb, rel, x, x0, win, wl, sw):
    grid = N_NODES // ROW_BLK
    row_spec = pl.BlockSpec((ROW_BLK, HIDDEN), lambda i: (i, 0))
    cnt_spec = pl.BlockSpec((ROW_BLK, N_REL), lambda i: (i, 0))
    full = lambda shape: pl.BlockSpec(shape, lambda i: (0, 0))
    return pl.pallas_call(
        _tc_layer_body,
        grid=(grid,),
        in_specs=[
            pl.BlockSpec(memory_space=pltpu.SMEM),
            row_spec, row_spec, cnt_spec, cnt_spec,
            full((N_REL, HIDDEN)),
            row_spec, row_spec,
            full((HIDDEN, HIDDEN)), full((HIDDEN, HIDDEN)),
        ],
        out_specs=row_spec,
        out_shape=jax.ShapeDtypeStruct((N_NODES, HIDDEN), jnp.float32),
    )(sw, pa, pb, ca, cb, rel, x, x0, win, wl)


def kernel(entity_emb, rel_emb, W_in, W_loop, skip_weights, edge_index, edge_type):
    src = edge_index[0].astype(jnp.int32)
    dst = edge_index[1].astype(jnp.int32)
    et = edge_type.astype(jnp.int32)

    cnt2 = _sc_hist(dst, et).reshape(NUM_CORES, N_NODES, N_REL)
    ca = cnt2[0]
    cb = cnt2[1]

    x0 = entity_emb
    x = x0
    for i in range(N_LAYERS):
        parts = _sc_agg(src, dst, x)
        x = _tc_layer(parts[0, :N_NODES], parts[1, :N_NODES], ca, cb, rel_emb,
                      x, x0, W_in[i], W_loop[i], skip_weights[i].reshape(1))
    return x


# speedup vs baseline: 9.4314x; 1.4470x over previous
import functools

import jax
import jax.numpy as jnp
from jax import lax
from jax.experimental import pallas as pl
from jax.experimental.pallas import tpu as pltpu
from jax.experimental.pallas import tpu_sc as plsc

N_NODES = 10000
N_EDGES = 320000
N_REL = 64
HIDDEN = 128
N_LAYERS = 3

NUM_CORES = 2
NUM_SUBCORES = 16
NW = NUM_CORES * NUM_SUBCORES

CHUNK = 128
TOTAL_CHUNKS = N_EDGES // CHUNK
BASE_CHUNKS = TOTAL_CHUNKS // NW
EXTRA_W = TOTAL_CHUNKS % NW

N_PAD = 10240
ROWS_PER_SUB = N_PAD // NUM_SUBCORES
ZROWS = 128

CNT_TOT = N_NODES * N_REL
CNT_SUBS = 8
CNT_PER_SUB = CNT_TOT // CNT_SUBS
ZCNT = 16000

_mesh = plsc.VectorSubcoreMesh(core_axis_name="c", subcore_axis_name="s")


@functools.partial(
    pl.kernel,
    out_type=jax.ShapeDtypeStruct((NUM_CORES * CNT_TOT,), jnp.float32),
    mesh=_mesh,
    scratch_types=[
        pltpu.VMEM_SHARED((CNT_TOT,), jnp.float32),
        pltpu.VMEM((CHUNK,), jnp.int32),
        pltpu.VMEM((CHUNK,), jnp.int32),
        pltpu.VMEM((CHUNK,), jnp.int32),
        pltpu.VMEM((CHUNK,), jnp.float32),
        pltpu.VMEM((ZCNT,), jnp.float32),
    ],
)
def _sc_hist(dst_hbm, et_hbm, cnt_out, cnt_sh, dst_v, et_v, flat_v, ones_v, zero_v):
    cid = lax.axis_index("c")
    sid = lax.axis_index("s")
    wid = sid * NUM_CORES + cid

    def zstore(i, carry):
        zero_v[pl.ds(i * 16, 16)] = jnp.zeros((16,), jnp.float32)
        return carry
    lax.fori_loop(0, ZCNT // 16, zstore, 0)
    for k in range(CHUNK // 16):
        ones_v[pl.ds(k * 16, 16)] = jnp.ones((16,), jnp.float32)

    @pl.when(sid < CNT_SUBS)
    def _():
        for k in range(CNT_PER_SUB // ZCNT):
            pltpu.sync_copy(zero_v, cnt_sh.at[pl.ds(sid * CNT_PER_SUB + k * ZCNT, ZCNT)])
    plsc.subcore_barrier()

    nj = BASE_CHUNKS + jnp.where(wid < EXTRA_W, 1, 0)

    def chunk_body(t, carry):
        off = pl.multiple_of((wid + t * NW) * CHUNK, CHUNK)
        pltpu.sync_copy(dst_hbm.at[pl.ds(off, CHUNK)], dst_v)
        pltpu.sync_copy(et_hbm.at[pl.ds(off, CHUNK)], et_v)
        for k in range(CHUNK // 16):
            d16 = dst_v[pl.ds(k * 16, 16)]
            e16 = et_v[pl.ds(k * 16, 16)]
            flat_v[pl.ds(k * 16, 16)] = d16 * N_REL + e16
        pltpu.sync_copy(ones_v, cnt_sh.at[flat_v], add=True)
        return carry
    lax.fori_loop(0, nj, chunk_body, 0)
    plsc.subcore_barrier()

    @pl.when(sid < CNT_SUBS)
    def _():
        for k in range(CNT_PER_SUB // ZCNT):
            base = sid * CNT_PER_SUB + k * ZCNT
            pltpu.sync_copy(cnt_sh.at[pl.ds(base, ZCNT)],
                            cnt_out.at[pl.ds(cid * CNT_TOT + base, ZCNT)])


GROUPS = BASE_CHUNKS // 2


@functools.partial(
    pl.kernel,
    out_type=jax.ShapeDtypeStruct((NUM_CORES, N_PAD, HIDDEN), jnp.float32),
    mesh=_mesh,
    scratch_types=[
        pltpu.VMEM_SHARED((N_PAD, HIDDEN), jnp.float32),
        pltpu.VMEM((CHUNK,), jnp.int32),
        pltpu.VMEM((CHUNK,), jnp.int32),
        pltpu.VMEM((CHUNK,), jnp.int32),
        pltpu.VMEM((CHUNK,), jnp.int32),
        pltpu.VMEM((CHUNK, HIDDEN), jnp.float32),
        pltpu.VMEM((CHUNK, HIDDEN), jnp.float32),
        pltpu.SemaphoreType.DMA,
        pltpu.SemaphoreType.DMA,
    ],
)
def _sc_agg(src_hbm, dst_hbm, x_hbm, out_hbm, agg_sh,
            src0, dst0, src1, dst1, rows0, rows1, sem0, sem1):
    cid = lax.axis_index("c")
    sid = lax.axis_index("s")
    wid = sid * NUM_CORES + cid

    def zrow(r, carry):
        for c in range(HIDDEN // 16):
            rows0[r, pl.ds(c * 16, 16)] = jnp.zeros((16,), jnp.float32)
        return carry
    lax.fori_loop(0, ZROWS, zrow, 0)
    for k in range(ROWS_PER_SUB // ZROWS):
        pltpu.sync_copy(rows0, agg_sh.at[pl.ds(sid * ROWS_PER_SUB + k * ZROWS, ZROWS), :])
    plsc.subcore_barrier()

    def load_idx(t, sv, dv):
        off = pl.multiple_of((wid + t * NW) * CHUNK, CHUNK)
        pltpu.sync_copy(src_hbm.at[pl.ds(off, CHUNK)], sv)
        pltpu.sync_copy(dst_hbm.at[pl.ds(off, CHUNK)], dv)

    load_idx(0, src0, dst0)
    pltpu.async_copy(x_hbm.at[src0], rows0, sem0)

    def group_body(g, carry):
        load_idx(2 * g + 1, src1, dst1)
        pltpu.async_copy(x_hbm.at[src1], rows1, sem1)
        pltpu.make_async_copy(x_hbm.at[src0], rows0, sem0).wait()
        pltpu.sync_copy(rows0, agg_sh.at[dst0], add=True)

        @pl.when(g < GROUPS - 1)
        def _():
            load_idx(2 * g + 2, src0, dst0)
            pltpu.async_copy(x_hbm.at[src0], rows0, sem0)
        pltpu.make_async_copy(x_hbm.at[src1], rows1, sem1).wait()
        pltpu.sync_copy(rows1, agg_sh.at[dst1], add=True)
        return carry
    lax.fori_loop(0, GROUPS, group_body, 0)

    @pl.when(wid < EXTRA_W)
    def _():
        load_idx(BASE_CHUNKS, src0, dst0)
        pltpu.async_copy(x_hbm.at[src0], rows0, sem0).wait()
        pltpu.sync_copy(rows0, agg_sh.at[dst0], add=True)
    plsc.subcore_barrier()

    for k in range(ROWS_PER_SUB // ZROWS):
        sl = pl.ds(sid * ROWS_PER_SUB + k * ZROWS, ZROWS)
        pltpu.sync_copy(agg_sh.at[sl, :], out_hbm.at[cid, sl, :])


ROW_BLK = 1000


def _tc_layer_body(sw_ref, pa_ref, pb_ref, ca_ref, cb_ref, rel_ref, x_ref,
                   x0_ref, win_ref, wl_ref, o_ref):
    cnt = ca_ref[...] + cb_ref[...]
    deg = jnp.sum(cnt, axis=1, keepdims=True)
    inv = 1.0 / jnp.maximum(deg, 1.0)
    rel_term = jnp.dot(cnt, rel_ref[...], preferred_element_type=jnp.float32)
    agg = (pa_ref[...] + pb_ref[...] - rel_term) * inv
    h = (jnp.dot(agg, win_ref[...], preferred_element_type=jnp.float32)
         + jnp.dot(x_ref[...], wl_ref[...], preferred_element_type=jnp.float32))
    h = jnp.where(h >= 0, h, 0.2 * h)
    alpha = 1.0 / (1.0 + jnp.exp(-sw_ref[0]))
    o_ref[...] = (1.0 - alpha) * h + alpha * x0_ref[...]


def _tc_layer(pa, pb, ca, cb, rel, x, x0, win, wl, sw):
    grid = N_NODES // ROW_BLK
    row_spec = pl.BlockSpec((ROW_BLK, HIDDEN), lambda i: (i, 0))
    cnt_spec = pl.BlockSpec((ROW_BLK, N_REL), lambda i: (i, 0))
    full = lambda shape: pl.BlockSpec(shape, lambda i: (0, 0))
    return pl.pallas_call(
        _tc_layer_body,
        grid=(grid,),
        in_specs=[
            pl.BlockSpec(memory_space=pltpu.SMEM),
            row_spec, row_spec, cnt_spec, cnt_spec,
            full((N_REL, HIDDEN)),
            row_spec, row_spec,
            full((HIDDEN, HIDDEN)), full((HIDDEN, HIDDEN)),
        ],
        out_specs=row_spec,
        out_shape=jax.ShapeDtypeStruct((N_NODES, HIDDEN), jnp.float32),
    )(sw, pa, pb, ca, cb, rel, x, x0, win, wl)


def kernel(entity_emb, rel_emb, W_in, W_loop, skip_weights, edge_index, edge_type):
    src = edge_index[0].astype(jnp.int32)
    dst = edge_index[1].astype(jnp.int32)
    et = edge_type.astype(jnp.int32)

    cnt2 = _sc_hist(dst, et).reshape(NUM_CORES, N_NODES, N_REL)
    ca = cnt2[0]
    cb = cnt2[1]

    x0 = entity_emb
    x = x0
    for i in range(N_LAYERS):
        parts = _sc_agg(src, dst, x)
        x = _tc_layer(parts[0, :N_NODES], parts[1, :N_NODES], ca, cb, rel_emb,
                      x, x0, W_in[i], W_loop[i], skip_weights[i].reshape(1))
    return x
